# baseline (speedup 1.0000x reference)
import functools

import jax
import jax.numpy as jnp
from jax import lax
from jax.experimental import pallas as pl
from jax.experimental.pallas import tpu as pltpu
from jax.experimental.pallas import tpu_sc as plsc

N = 10000
E = 320000
D_IN = 16
D_H = 128

NC = 2
NS = 16
NW = NC * NS
L = 16
NST = N // NS

CB = 128
EPT = 10240
CH = EPT // CB
CHP = CH + 2
EP = NW * EPT

NPAD = 10240
RW = NPAD // NS

DH2 = D_H // NC
CH2 = EP // (NS * CB)
CHP2 = CH2 + 2

PK_SHIFT = 14
PK_MASK = (1 << PK_SHIFT) - 1

_MESH = plsc.VectorSubcoreMesh(
    core_axis_name="c", subcore_axis_name="s", num_cores=NC, num_subcores=NS)


def _unpack_src(idxp_v, j, out_v):
    for k in range(CB // L):
        p = idxp_v[j, pl.ds(k * L, L)]
        out_v[pl.ds(k * L, L)] = jnp.bitwise_and(p, PK_MASK)


def _unpack_dst(idxp_v, j, out_v):
    for k in range(CB // L):
        p = idxp_v[j, pl.ds(k * L, L)]
        out_v[pl.ds(k * L, L)] = lax.shift_right_logical(p, PK_SHIFT)


def _rsqrt_vec(x):
    xi = lax.bitcast_convert_type(x, jnp.int32)
    yi = 0x5F3759DF - lax.shift_right_logical(xi, 1)
    y = lax.bitcast_convert_type(yi, jnp.float32)
    for _ in range(3):
        y = y * (1.5 - 0.5 * x * y * y)
    return y


@functools.partial(
    pl.kernel,
    out_type=[
        jax.ShapeDtypeStruct((NC, NPAD, D_IN), jnp.float32),
        jax.ShapeDtypeStruct((NC, NPAD, D_IN), jnp.float32),
    ],
    mesh=_MESH,
    compiler_params=pltpu.CompilerParams(use_tc_tiling_on_sc=False),
    scratch_types=[
        pltpu.VMEM((CHP2, CB), jnp.int32),
        pltpu.VMEM((CB,), jnp.int32),
        pltpu.VMEM((CB,), jnp.int32),
        pltpu.VMEM((CB,), jnp.int32),
        pltpu.VMEM((CB, D_IN), jnp.float32),
        pltpu.VMEM((CB, D_IN), jnp.float32),
        pltpu.VMEM((CB, D_IN), jnp.float32),
        pltpu.VMEM((L, D_IN), jnp.float32),
        pltpu.VMEM((RW, D_IN), jnp.float32),
        pltpu.VMEM((RW, D_IN), jnp.float32),
        pltpu.VMEM_SHARED((NPAD, D_IN), jnp.float32),
        pltpu.VMEM_SHARED((NPAD, D_IN), jnp.float32),
        pltpu.VMEM_SHARED((NPAD, D_IN), jnp.float32),
        pltpu.SemaphoreType.DMA,
        pltpu.SemaphoreType.DMA,
    ],
)
def _pre_kernel(ea_hbm, pk2_hbm, pk_hbm, acc_out, dis_out,
                idxp_v, sidx0_v, sidx1_v, didx_v, ones_v, rows0_v, rows1_v,
                zrows_v, drep_v, erows_v, shared_deg, shared_table, shared_acc,
                sem0, sem1):
    c = lax.axis_index("c")
    s = lax.axis_index("s")
    wid = c * NS + s
    sidxs = (sidx0_v, sidx1_v)
    rows = (rows0_v, rows1_v)
    sems = (sem0, sem1)

    for i in range(CB):
        ones_v[i, pl.ds(0, D_IN)] = jnp.ones((L,), jnp.float32)
    for i in range(L):
        zrows_v[i, pl.ds(0, D_IN)] = jnp.zeros((L,), jnp.float32)

    def zinit(i, carry):
        pltpu.sync_copy(zrows_v, shared_deg.at[pl.ds(s * RW + i * L, L)])
        pltpu.sync_copy(zrows_v, shared_acc.at[pl.ds(s * RW + i * L, L)])
        return carry

    lax.fori_loop(0, RW // L, zinit, 0)
    plsc.subcore_barrier()

    pltpu.sync_copy(pk2_hbm.at[s], idxp_v)

    def deg_body(j, carry):
        _unpack_dst(idxp_v, j, didx_v)
        pltpu.sync_copy(ones_v, shared_deg.at[didx_v], add=True)
        return carry

    lax.fori_loop(0, CH2, deg_body, 0)
    plsc.subcore_barrier()

    pltpu.sync_copy(shared_deg.at[pl.ds(s * RW, RW)], drep_v)

    def dis_body(r, carry):
        d = drep_v[r, pl.ds(0, D_IN)]
        drep_v[r, pl.ds(0, D_IN)] = _rsqrt_vec(d + 1.0)
        return carry

    lax.fori_loop(0, RW, dis_body, 0)
    pltpu.sync_copy(drep_v, dis_out.at[c].at[pl.ds(s * RW, RW)])
    pltpu.sync_copy(ea_hbm.at[pl.ds(s * RW, RW)], erows_v)

    def scale_body(r, carry):
        erows_v[r, pl.ds(0, D_IN)] = (erows_v[r, pl.ds(0, D_IN)]
                                      * drep_v[r, pl.ds(0, D_IN)])
        return carry

    lax.fori_loop(0, RW, scale_body, 0)
    pltpu.sync_copy(erows_v, shared_table.at[pl.ds(s * RW, RW)])
    plsc.subcore_barrier()

    pltpu.sync_copy(pk_hbm.at[wid], idxp_v.at[pl.ds(0, CHP)])

    for b in range(2):
        _unpack_src(idxp_v, b, sidxs[b])
        pltpu.async_copy(shared_table.at[sidxs[b]], rows[b], sems[b])

    def body(i, carry):
        for b in range(2):
            j = 2 * i + b
            pltpu.make_async_copy(shared_table.at[sidxs[b]],
                                  rows[b], sems[b]).wait()
            _unpack_dst(idxp_v, j, didx_v)
            pltpu.sync_copy(rows[b], shared_acc.at[didx_v], add=True)
            _unpack_src(idxp_v, j + 2, sidxs[b])
            pltpu.async_copy(shared_table.at[sidxs[b]], rows[b], sems[b])
        return carry

    lax.fori_loop(0, CH // 2, body, 0)
    for b in range(2):
        pltpu.make_async_copy(shared_table.at[sidxs[b]],
                              rows[b], sems[b]).wait()
    plsc.subcore_barrier()
    pltpu.sync_copy(shared_acc.at[pl.ds(s * RW, RW)],
                    acc_out.at[c].at[pl.ds(s * RW, RW)])


def _make_conv(d_feat, feature_split):
    chunks = CH2 if feature_split else CH
    chunks_p = CHP2 if feature_split else CHP
    table_shape = (NC, N, d_feat) if feature_split else (N, d_feat)

    @functools.partial(
        pl.kernel,
        out_type=jax.ShapeDtypeStruct((NC, NPAD, d_feat), jnp.float32),
        mesh=_MESH,
        compiler_params=pltpu.CompilerParams(use_tc_tiling_on_sc=False),
        scratch_types=[
            pltpu.VMEM((chunks_p, CB), jnp.int32),
            pltpu.VMEM((CB,), jnp.int32),
            pltpu.VMEM((CB,), jnp.int32),
            pltpu.VMEM((CB,), jnp.int32),
            pltpu.VMEM((CB, d_feat), jnp.float32),
            pltpu.VMEM((CB, d_feat), jnp.float32),
            pltpu.VMEM((L, d_feat), jnp.float32),
            pltpu.VMEM_SHARED((N, d_feat), jnp.float32),
            pltpu.VMEM_SHARED((NPAD, d_feat), jnp.float32),
            pltpu.SemaphoreType.DMA,
            pltpu.SemaphoreType.DMA,
        ],
    )
    def _conv(table_hbm, pk_hbm, out_hbm,
              idxp_v, sidx0_v, sidx1_v, didx_v, rows0_v, rows1_v, zrows_v,
              shared_table, shared_acc, sem0, sem1):
        c = lax.axis_index("c")
        s = lax.axis_index("s")
        table_src = table_hbm.at[c] if feature_split else table_hbm
        my_pk = pk_hbm.at[s] if feature_split else pk_hbm.at[c * NS + s]
        sidxs = (sidx0_v, sidx1_v)
        rows = (rows0_v, rows1_v)
        sems = (sem0, sem1)

        for i in range(L):
            for k in range(d_feat // L):
                zrows_v[i, pl.ds(k * L, L)] = jnp.zeros((L,), jnp.float32)

        def zinit(i, carry):
            pltpu.sync_copy(zrows_v, shared_acc.at[pl.ds(s * RW + i * L, L)])
            return carry

        lax.fori_loop(0, RW // L, zinit, 0)
        pltpu.sync_copy(table_src.at[pl.ds(s * NST, NST)],
                        shared_table.at[pl.ds(s * NST, NST)])
        plsc.subcore_barrier()

        pltpu.sync_copy(my_pk, idxp_v)

        for b in range(2):
            _unpack_src(idxp_v, b, sidxs[b])
            pltpu.async_copy(shared_table.at[sidxs[b]], rows[b], sems[b])

        def body(i, carry):
            for b in range(2):
                j = 2 * i + b
                pltpu.make_async_copy(shared_table.at[sidxs[b]],
                                      rows[b], sems[b]).wait()
                _unpack_dst(idxp_v, j, didx_v)
                pltpu.sync_copy(rows[b], shared_acc.at[didx_v], add=True)
                _unpack_src(idxp_v, j + 2, sidxs[b])
                pltpu.async_copy(shared_table.at[sidxs[b]], rows[b], sems[b])
            return carry

        lax.fori_loop(0, chunks // 2, body, 0)
        for b in range(2):
            pltpu.make_async_copy(shared_table.at[sidxs[b]],
                                  rows[b], sems[b]).wait()
        plsc.subcore_barrier()
        pltpu.sync_copy(shared_acc.at[pl.ds(s * RW, RW)],
                        out_hbm.at[c].at[pl.ds(s * RW, RW)])

    return _conv


_conv2 = _make_conv(DH2, feature_split=True)


def _tc2_body(acc_ref, ea_ref, disv_ref, w1_ref, b1_ref, w2_ref, g2_ref):
    dis = disv_ref[0, :N, 0:1]
    g1 = dis * ea_ref[...]
    a = acc_ref[0, :N, :] + acc_ref[1, :N, :] + g1
    z1 = dis * a
    h1 = jnp.maximum(
        jnp.dot(z1, w1_ref[...], preferred_element_type=jnp.float32)
        + b1_ref[...], 0.0)
    hw2 = jnp.dot(h1, w2_ref[...], preferred_element_type=jnp.float32)
    g2 = dis * hw2
    g2_ref[0, :, :] = g2[:, :DH2]
    g2_ref[1, :, :] = g2[:, DH2:]


def _tc3_body(acc_ref, g2_ref, disv_ref, b2_ref, fc1w_ref, fc1b_ref,
              fc2w_ref, fc2b_ref, out_ref):
    dis = disv_ref[0, :N, 0:1]
    a = jnp.concatenate(
        [acc_ref[0, :N, :] + g2_ref[0, :, :],
         acc_ref[1, :N, :] + g2_ref[1, :, :]], axis=1)
    h = jnp.maximum(dis * a + b2_ref[...], 0.0)
    pooled = jnp.sum(h, axis=0, keepdims=True) * (1.0 / N)
    h2 = jnp.maximum(
        jnp.dot(pooled, fc1w_ref[...], preferred_element_type=jnp.float32)
        + fc1b_ref[...], 0.0)
    out_ref[...] = (
        jnp.dot(h2, fc2w_ref[...], preferred_element_type=jnp.float32)
        + fc2b_ref[...])


def kernel(x, edge_index, edge_attr, W1, b1, W2, b2, fc1_w, fc1_b, fc2_w, fc2_b):
    del x
    src = edge_index[0].astype(jnp.int32)
    dst = edge_index[1].astype(jnp.int32)
    npad_e = EP - E
    pad_src = (jnp.arange(npad_e, dtype=jnp.int32) % N)
    pad_dst = N + (jnp.arange(npad_e, dtype=jnp.int32) % (NPAD - N))
    pk_f = jnp.bitwise_or(jnp.concatenate([src, pad_src]),
                          jnp.left_shift(jnp.concatenate([dst, pad_dst]),
                                         PK_SHIFT))
    extra = jnp.full((NW, CHP - CH, CB), N << PK_SHIFT, jnp.int32)
    pk_p = jnp.concatenate([pk_f.reshape(NW, CH, CB), extra], axis=1)
    extra2 = jnp.full((NS, CHP2 - CH2, CB), N << PK_SHIFT, jnp.int32)
    pk_p2 = jnp.concatenate([pk_f.reshape(NS, CH2, CB), extra2], axis=1)

    ea_pad = jnp.pad(edge_attr, ((0, NPAD - N), (0, 0)))

    acc1, disv = _pre_kernel(ea_pad, pk_p2, pk_p)

    g2 = pl.pallas_call(
        _tc2_body,
        out_shape=jax.ShapeDtypeStruct((NC, N, DH2), jnp.float32),
    )(acc1, edge_attr, disv, W1, b1.reshape(1, D_H), W2)

    acc2 = _conv2(g2, pk_p2)

    out = pl.pallas_call(
        _tc3_body,
        out_shape=jax.ShapeDtypeStruct((1, 2), jnp.float32),
    )(acc2, g2, disv, b2.reshape(1, D_H), fc1_w, fc1_b.reshape(1, D_H),
      fc2_w, fc2_b.reshape(1, 2))
    return out


# speedup vs baseline: 1.1620x; 1.1620x over previous
import functools

import jax
import jax.numpy as jnp
from jax import lax
from jax.experimental import pallas as pl
from jax.experimental.pallas import tpu as pltpu
from jax.experimental.pallas import tpu_sc as plsc

N = 10000
E = 320000
D_IN = 16
D_H = 128

NC = 2
NS = 16
NW = NC * NS
L = 16

CB = 128
CH = 79
EPT = CH * CB
EP = NW * EPT
CH2 = 2 * CH

NPAD = 10240
RW = NPAD // NS
DH2 = D_H // NC

PK_SHIFT = 14
PK_MASK = (1 << PK_SHIFT) - 1

_MESH = plsc.VectorSubcoreMesh(
    core_axis_name="c", subcore_axis_name="s", num_cores=NC, num_subcores=NS)


def _unpack_src(idxp_v, j, out_v):
    for k in range(CB // L):
        p = idxp_v[j, pl.ds(k * L, L)]
        out_v[pl.ds(k * L, L)] = jnp.bitwise_and(p, PK_MASK)


def _unpack_dst(idxp_v, j, out_v):
    for k in range(CB // L):
        p = idxp_v[j, pl.ds(k * L, L)]
        out_v[pl.ds(k * L, L)] = lax.shift_right_logical(p, PK_SHIFT)


@functools.partial(
    pl.kernel,
    out_type=jax.ShapeDtypeStruct((NC, NPAD), jnp.float32),
    mesh=_MESH,
    compiler_params=pltpu.CompilerParams(use_tc_tiling_on_sc=False),
    scratch_types=[
        pltpu.VMEM((CH, CB), jnp.int32),
        pltpu.VMEM((CB,), jnp.int32),
        pltpu.VMEM((CB,), jnp.float32),
        pltpu.VMEM((RW,), jnp.float32),
        pltpu.VMEM_SHARED((NPAD,), jnp.float32),
    ],
)
def _deg_kernel(pk_hbm, out_hbm, idxp_v, didx_v, ones_v, zrow_v, shared_deg):
    c = lax.axis_index("c")
    s = lax.axis_index("s")
    wid = c * NS + s
    for i in range(CB // L):
        ones_v[pl.ds(i * L, L)] = jnp.ones((L,), jnp.float32)
    for i in range(RW // L):
        zrow_v[pl.ds(i * L, L)] = jnp.zeros((L,), jnp.float32)
    pltpu.sync_copy(zrow_v, shared_deg.at[pl.ds(s * RW, RW)])
    plsc.subcore_barrier()
    pltpu.sync_copy(pk_hbm.at[wid], idxp_v)

    def body(j, carry):
        _unpack_dst(idxp_v, j, didx_v)
        pltpu.sync_copy(ones_v, shared_deg.at[didx_v], add=True)
        return carry

    lax.fori_loop(0, CH, body, 0)
    plsc.subcore_barrier()
    pltpu.sync_copy(shared_deg.at[pl.ds(s * RW, RW)],
                    out_hbm.at[c].at[pl.ds(s * RW, RW)])


def _make_conv(d_feat, feature_split):
    chunks = CH2 if feature_split else CH
    nst = N // NS

    @functools.partial(
        pl.kernel,
        out_type=jax.ShapeDtypeStruct((NC, NPAD, d_feat), jnp.float32),
        mesh=_MESH,
        compiler_params=pltpu.CompilerParams(use_tc_tiling_on_sc=False),
        scratch_types=[
            pltpu.VMEM((chunks, CB), jnp.int32),
            pltpu.VMEM((CB,), jnp.int32),
            pltpu.VMEM((CB,), jnp.int32),
            pltpu.VMEM((CB,), jnp.int32),
            pltpu.VMEM((CB, d_feat), jnp.float32),
            pltpu.VMEM((CB, d_feat), jnp.float32),
            pltpu.VMEM((L, d_feat), jnp.float32),
            pltpu.VMEM_SHARED((N, d_feat), jnp.float32),
            pltpu.VMEM_SHARED((NPAD, d_feat), jnp.float32),
            pltpu.SemaphoreType.DMA,
            pltpu.SemaphoreType.DMA,
        ],
    )
    def _conv(table_hbm, pk_hbm, out_hbm,
              idxp_v, sidx0_v, sidx1_v, didx_v, rows0_v, rows1_v, zrows_v,
              shared_table, shared_acc, sem0, sem1):
        c = lax.axis_index("c")
        s = lax.axis_index("s")
        table_src = table_hbm.at[c] if feature_split else table_hbm
        my_pk = pk_hbm.at[s] if feature_split else pk_hbm.at[c * NS + s]
        sidxs = (sidx0_v, sidx1_v)
        rows = (rows0_v, rows1_v)
        sems = (sem0, sem1)

        for i in range(L):
            for k in range(d_feat // L):
                zrows_v[i, pl.ds(k * L, L)] = jnp.zeros((L,), jnp.float32)

        def zinit(i, carry):
            pltpu.sync_copy(zrows_v, shared_acc.at[pl.ds(s * RW + i * L, L)])
            return carry

        lax.fori_loop(0, RW // L, zinit, 0)
        pltpu.sync_copy(table_src.at[pl.ds(s * nst, nst)],
                        shared_table.at[pl.ds(s * nst, nst)])
        plsc.subcore_barrier()

        pltpu.sync_copy(my_pk, idxp_v)

        for b in range(2):
            _unpack_src(idxp_v, b, sidxs[b])
            pltpu.async_copy(shared_table.at[sidxs[b]], rows[b], sems[b])

        def body(i, carry):
            for b in range(2):
                j = 2 * i + b
                pltpu.make_async_copy(shared_table.at[sidxs[b]],
                                      rows[b], sems[b]).wait()
                _unpack_dst(idxp_v, j, didx_v)
                pltpu.sync_copy(rows[b], shared_acc.at[didx_v], add=True)
                _unpack_src(idxp_v, lax.rem(j + 2, chunks), sidxs[b])
                pltpu.async_copy(shared_table.at[sidxs[b]], rows[b], sems[b])
            return carry

        lax.fori_loop(0, chunks // 2, body, 0)
        if chunks % 2:
            pltpu.make_async_copy(shared_table.at[sidxs[0]],
                                  rows[0], sems[0]).wait()
            _unpack_dst(idxp_v, chunks - 1, didx_v)
            pltpu.sync_copy(rows[0], shared_acc.at[didx_v], add=True)
            pltpu.async_copy(shared_table.at[sidxs[0]], rows[0], sems[0])
        for b in range(2):
            pltpu.make_async_copy(shared_table.at[sidxs[b]],
                                  rows[b], sems[b]).wait()
        plsc.subcore_barrier()
        pltpu.sync_copy(shared_acc.at[pl.ds(s * RW, RW)],
                        out_hbm.at[c].at[pl.ds(s * RW, RW)])

    return _conv


_conv1 = _make_conv(D_IN, feature_split=False)
_conv2 = _make_conv(DH2, feature_split=True)


def _tc1_body(degp_ref, ea_ref, dis_ref, g1_ref):
    deg = degp_ref[0, :N] + degp_ref[1, :N] + 1.0
    dis = lax.rsqrt(deg)
    dis_ref[...] = dis[:, None]
    g1_ref[...] = dis[:, None] * ea_ref[...]


def _tc2_body(acc_ref, g1_ref, dis_ref, w1_ref, b1_ref, w2_ref, g2_ref):
    a = acc_ref[0, :N, :] + acc_ref[1, :N, :] + g1_ref[...]
    z1 = dis_ref[...] * a
    h1 = jnp.maximum(
        jnp.dot(z1, w1_ref[...], preferred_element_type=jnp.float32)
        + b1_ref[...], 0.0)
    hw2 = jnp.dot(h1, w2_ref[...], preferred_element_type=jnp.float32)
    g2 = dis_ref[...] * hw2
    g2_ref[0, :, :] = g2[:, :DH2]
    g2_ref[1, :, :] = g2[:, DH2:]


def _tc3_body(acc_ref, g2_ref, dis_ref, b2_ref, fc1w_ref, fc1b_ref,
              fc2w_ref, fc2b_ref, out_ref):
    a = jnp.concatenate(
        [acc_ref[0, :N, :] + g2_ref[0, :, :],
         acc_ref[1, :N, :] + g2_ref[1, :, :]], axis=1)
    h = jnp.maximum(dis_ref[...] * a + b2_ref[...], 0.0)
    pooled = jnp.sum(h, axis=0, keepdims=True) * (1.0 / N)
    h2 = jnp.maximum(
        jnp.dot(pooled, fc1w_ref[...], preferred_element_type=jnp.float32)
        + fc1b_ref[...], 0.0)
    out_ref[...] = (
        jnp.dot(h2, fc2w_ref[...], preferred_element_type=jnp.float32)
        + fc2b_ref[...])


def kernel(x, edge_index, edge_attr, W1, b1, W2, b2, fc1_w, fc1_b, fc2_w, fc2_b):
    del x
    src = edge_index[0].astype(jnp.int32)
    dst = edge_index[1].astype(jnp.int32)
    npad_e = EP - E
    pad_src = (jnp.arange(npad_e, dtype=jnp.int32) % N)
    pad_dst = N + (jnp.arange(npad_e, dtype=jnp.int32) % (NPAD - N))
    pk_f = jnp.bitwise_or(jnp.concatenate([src, pad_src]),
                          jnp.left_shift(jnp.concatenate([dst, pad_dst]),
                                         PK_SHIFT))
    pk_p = pk_f.reshape(NW, CH, CB)
    pk_p2 = pk_f.reshape(NS, CH2, CB)

    degp = _deg_kernel(pk_p)

    dis, g1 = pl.pallas_call(
        _tc1_body,
        out_shape=[
            jax.ShapeDtypeStruct((N, 1), jnp.float32),
            jax.ShapeDtypeStruct((N, D_IN), jnp.float32),
        ],
    )(degp, edge_attr)

    acc1 = _conv1(g1, pk_p)

    g2 = pl.pallas_call(
        _tc2_body,
        out_shape=jax.ShapeDtypeStruct((NC, N, DH2), jnp.float32),
    )(acc1, g1, dis, W1, b1.reshape(1, D_H), W2)

    acc2 = _conv2(g2, pk_p2)

    out = pl.pallas_call(
        _tc3_body,
        out_shape=jax.ShapeDtypeStruct((1, 2), jnp.float32),
    )(acc2, g2, dis, b2.reshape(1, D_H), fc1_w, fc1_b.reshape(1, D_H),
      fc2_w, fc2_b.reshape(1, 2))
    return out
